# baseline (speedup 1.0000x reference)
import functools
import math

import jax
import jax.numpy as jnp
from jax.experimental import pallas as pl
from jax.experimental.pallas import tpu as pltpu

INPUT_DIM = 128
HIDDEN = 128
KEY_DIM = 64
MAX_SLOTS = 65536
D_KEY = 64
D_VAL = 64
TEMP = 0.1
BATCH = 512
SEQ = 50
EPS = 1e-8

SLOT_TILE = 8192
NUM_SLOT_TILES = MAX_SLOTS // SLOT_TILE


def _lstm_step_kernel(x_ref, w_ref, b_ref, out_ref, xh_ref, c_ref):
    t = pl.program_id(0)

    @pl.when(t == 0)
    def _init():
        xh_ref[...] = jnp.zeros_like(xh_ref)
        c_ref[...] = jnp.zeros_like(c_ref)

    x = x_ref[:, pl.ds(t * INPUT_DIM, INPUT_DIM)]
    xh_ref[:, :INPUT_DIM] = x.astype(jnp.bfloat16)
    gates = (
        jnp.dot(xh_ref[...], w_ref[...], preferred_element_type=jnp.float32)
        + b_ref[...]
    )
    def _sigmoid(x):
        return 0.5 * jnp.tanh(0.5 * x) + 0.5

    i = _sigmoid(gates[:, 0 * HIDDEN:1 * HIDDEN])
    f = _sigmoid(gates[:, 1 * HIDDEN:2 * HIDDEN])
    g = jnp.tanh(gates[:, 2 * HIDDEN:3 * HIDDEN])
    o = _sigmoid(gates[:, 3 * HIDDEN:4 * HIDDEN])
    c_new = f * c_ref[...] + i * g
    h_new = o * jnp.tanh(c_new)
    c_ref[...] = c_new
    xh_ref[:, INPUT_DIM:] = h_new.astype(jnp.bfloat16)

    @pl.when(t == SEQ - 1)
    def _fin():
        out_ref[...] = h_new


def _mem_read_kernel(cq_ref, kpw_ref, kpb_ref, keys_ref, vals_ref, fh_ref,
                     owh_ref, owc_ref, ob_ref, out_ref,
                     qa_ref, va_ref, acc_ref):
    s = pl.program_id(0)

    @pl.when(s == 0)
    def _init():
        q = (
            jnp.dot(cq_ref[...], kpw_ref[...], preferred_element_type=jnp.float32)
            + kpb_ref[...]
        )
        nrm = jnp.sqrt(jnp.sum(q * q, axis=-1, keepdims=True))
        qn = q / (nrm + EPS)
        qa_ref[...] = (qn * (math.log2(math.e) / TEMP)).astype(qa_ref.dtype)
        va_ref[:, D_VAL:] = jnp.ones((SLOT_TILE, 128 - D_VAL), jnp.bfloat16)
        acc_ref[...] = jnp.zeros_like(acc_ref)

    k = keys_ref[...]
    knrm = jnp.sqrt(jnp.sum(k * k, axis=-1, keepdims=True))
    kn = (k / (knrm + EPS)).astype(qa_ref.dtype)
    logits2 = jax.lax.dot_general(
        qa_ref[...], kn, (((1,), (1,)), ((), ())),
        preferred_element_type=jnp.float32,
    )
    p = jnp.exp2(logits2.astype(jnp.bfloat16))
    va_ref[:, :D_VAL] = vals_ref[...].astype(jnp.bfloat16)
    acc_ref[...] += jax.lax.dot_general(
        p, va_ref[...], (((1,), (0,)), ((), ())),
        preferred_element_type=jnp.float32,
    )

    @pl.when(s == NUM_SLOT_TILES - 1)
    def _fin():
        ctx = acc_ref[:, :D_VAL] / acc_ref[:, D_VAL:D_VAL + 1]
        out_ref[...] = (
            jnp.dot(fh_ref[...], owh_ref[...], preferred_element_type=jnp.float32)
            + jnp.dot(ctx, owc_ref[...], preferred_element_type=jnp.float32)
            + ob_ref[...]
        )


@functools.partial(jax.jit, static_argnames=())
def kernel(inputs, W_ih, W_hh, b_ih, b_hh, key_proj_W, key_proj_b,
           mem_keys, mem_values, out_W, out_b):
    f32 = jnp.float32
    bf16 = jnp.bfloat16
    w_cat = jnp.concatenate([W_ih.T, W_hh.T], axis=0).astype(bf16)
    bias = (b_ih + b_hh).reshape(1, 4 * HIDDEN)

    final_h = pl.pallas_call(
        _lstm_step_kernel,
        grid=(SEQ,),
        in_specs=[
            pl.BlockSpec((BATCH, SEQ * INPUT_DIM), lambda t: (0, 0)),
            pl.BlockSpec((INPUT_DIM + HIDDEN, 4 * HIDDEN), lambda t: (0, 0)),
            pl.BlockSpec((1, 4 * HIDDEN), lambda t: (0, 0)),
        ],
        out_specs=pl.BlockSpec((BATCH, HIDDEN), lambda t: (0, 0)),
        out_shape=jax.ShapeDtypeStruct((BATCH, HIDDEN), f32),
        scratch_shapes=[
            pltpu.VMEM((BATCH, INPUT_DIM + HIDDEN), bf16),
            pltpu.VMEM((BATCH, HIDDEN), f32),
        ],
    )(inputs.reshape(BATCH, SEQ * INPUT_DIM), w_cat, bias)

    cq = inputs[:, -1, :KEY_DIM]
    kpw_t = key_proj_W.T
    kpb = key_proj_b.reshape(1, D_KEY)
    owh_t = out_W[:, :HIDDEN].T
    owc_t = out_W[:, HIDDEN:].T
    ob = out_b.reshape(1, -1)

    logits = pl.pallas_call(
        _mem_read_kernel,
        grid=(NUM_SLOT_TILES,),
        in_specs=[
            pl.BlockSpec((BATCH, KEY_DIM), lambda s: (0, 0)),
            pl.BlockSpec((KEY_DIM, D_KEY), lambda s: (0, 0)),
            pl.BlockSpec((1, D_KEY), lambda s: (0, 0)),
            pl.BlockSpec((SLOT_TILE, D_KEY), lambda s: (s, 0)),
            pl.BlockSpec((SLOT_TILE, D_VAL), lambda s: (s, 0)),
            pl.BlockSpec((BATCH, HIDDEN), lambda s: (0, 0)),
            pl.BlockSpec((HIDDEN, out_W.shape[0]), lambda s: (0, 0)),
            pl.BlockSpec((D_VAL, out_W.shape[0]), lambda s: (0, 0)),
            pl.BlockSpec((1, out_W.shape[0]), lambda s: (0, 0)),
        ],
        out_specs=pl.BlockSpec((BATCH, out_W.shape[0]), lambda s: (0, 0)),
        out_shape=jax.ShapeDtypeStruct((BATCH, out_W.shape[0]), f32),
        scratch_shapes=[
            pltpu.VMEM((BATCH, D_KEY), jnp.float8_e4m3fn),
            pltpu.VMEM((SLOT_TILE, 128), bf16),
            pltpu.VMEM((BATCH, 128), f32),
        ],
    )(cq, kpw_t, kpb, mem_keys, mem_values, final_h, owh_t, owc_t, ob)

    return logits


# speedup vs baseline: 1.0114x; 1.0114x over previous
import functools
import math

import jax
import jax.numpy as jnp
from jax.experimental import pallas as pl
from jax.experimental.pallas import tpu as pltpu

INPUT_DIM = 128
HIDDEN = 128
KEY_DIM = 64
MAX_SLOTS = 65536
D_KEY = 64
D_VAL = 64
TEMP = 0.1
BATCH = 512
SEQ = 50
EPS = 1e-8

SLOT_TILE = 8192
NUM_SLOT_TILES = MAX_SLOTS // SLOT_TILE


def _lstm_step_kernel(x_ref, w_ref, b_ref, out_ref, xh_ref, c_ref):
    t = pl.program_id(0)

    @pl.when(t == 0)
    def _init():
        xh_ref[...] = jnp.zeros_like(xh_ref)
        c_ref[...] = jnp.zeros_like(c_ref)

    x = x_ref[:, pl.ds(t * INPUT_DIM, INPUT_DIM)]
    xh_ref[:, :INPUT_DIM] = x.astype(jnp.bfloat16)
    gates = (
        jnp.dot(xh_ref[...], w_ref[...], preferred_element_type=jnp.float32)
        + b_ref[...]
    )
    def _sigmoid(x):
        return 0.5 * jnp.tanh(0.5 * x) + 0.5

    i = _sigmoid(gates[:, 0 * HIDDEN:1 * HIDDEN])
    f = _sigmoid(gates[:, 1 * HIDDEN:2 * HIDDEN])
    g = jnp.tanh(gates[:, 2 * HIDDEN:3 * HIDDEN])
    o = _sigmoid(gates[:, 3 * HIDDEN:4 * HIDDEN])
    c_new = f * c_ref[...] + i * g
    h_new = o * jnp.tanh(c_new)
    c_ref[...] = c_new
    xh_ref[:, INPUT_DIM:] = h_new.astype(jnp.bfloat16)

    @pl.when(t == SEQ - 1)
    def _fin():
        out_ref[...] = h_new


def _mem_read_kernel(cq_ref, kpw_ref, kpb_ref, keys_ref, vals_ref, fh_ref,
                     owh_ref, owc_ref, ob_ref, out_ref,
                     qa_ref, va_ref, acc_ref):
    s = pl.program_id(0)

    @pl.when(s == 0)
    def _init():
        q = (
            jnp.dot(cq_ref[:, :KEY_DIM], kpw_ref[...],
                    preferred_element_type=jnp.float32)
            + kpb_ref[...]
        )
        nrm = jnp.sqrt(jnp.sum(q * q, axis=-1, keepdims=True))
        qn = q / (nrm + EPS)
        qa_ref[...] = (qn * (math.log2(math.e) / TEMP)).astype(qa_ref.dtype)
        va_ref[:, D_VAL:] = jnp.ones((SLOT_TILE, 128 - D_VAL), jnp.bfloat16)
        acc_ref[...] = jnp.zeros_like(acc_ref)

    k = keys_ref[...]
    knrm = jnp.sqrt(jnp.sum(k * k, axis=-1, keepdims=True))
    kn = (k / (knrm + EPS)).astype(qa_ref.dtype)
    logits2 = jax.lax.dot_general(
        qa_ref[...], kn, (((1,), (1,)), ((), ())),
        preferred_element_type=jnp.float32,
    )
    p = jnp.exp2(logits2.astype(jnp.bfloat16))
    va_ref[:, :D_VAL] = vals_ref[...].astype(jnp.bfloat16)
    acc_ref[...] += jax.lax.dot_general(
        p, va_ref[...], (((1,), (0,)), ((), ())),
        preferred_element_type=jnp.float32,
    )

    @pl.when(s == NUM_SLOT_TILES - 1)
    def _fin():
        ctx = acc_ref[:, :D_VAL] / acc_ref[:, D_VAL:D_VAL + 1]
        out_ref[...] = (
            jnp.dot(fh_ref[...], owh_ref[...], preferred_element_type=jnp.float32)
            + jnp.dot(ctx, owc_ref[...], preferred_element_type=jnp.float32)
            + ob_ref[...]
        )


@functools.partial(jax.jit, static_argnames=())
def kernel(inputs, W_ih, W_hh, b_ih, b_hh, key_proj_W, key_proj_b,
           mem_keys, mem_values, out_W, out_b):
    f32 = jnp.float32
    bf16 = jnp.bfloat16
    w_cat = jnp.concatenate([W_ih.T, W_hh.T], axis=0).astype(bf16)
    bias = (b_ih + b_hh).reshape(1, 4 * HIDDEN)

    final_h = pl.pallas_call(
        _lstm_step_kernel,
        grid=(SEQ,),
        in_specs=[
            pl.BlockSpec((BATCH, SEQ * INPUT_DIM), lambda t: (0, 0)),
            pl.BlockSpec((INPUT_DIM + HIDDEN, 4 * HIDDEN), lambda t: (0, 0)),
            pl.BlockSpec((1, 4 * HIDDEN), lambda t: (0, 0)),
        ],
        out_specs=pl.BlockSpec((BATCH, HIDDEN), lambda t: (0, 0)),
        out_shape=jax.ShapeDtypeStruct((BATCH, HIDDEN), f32),
        scratch_shapes=[
            pltpu.VMEM((BATCH, INPUT_DIM + HIDDEN), bf16),
            pltpu.VMEM((BATCH, HIDDEN), f32),
        ],
    )(inputs.reshape(BATCH, SEQ * INPUT_DIM), w_cat, bias)

    kpw_t = key_proj_W.T
    kpb = key_proj_b.reshape(1, D_KEY)
    owh_t = out_W[:, :HIDDEN].T
    owc_t = out_W[:, HIDDEN:].T
    ob = out_b.reshape(1, -1)

    logits = pl.pallas_call(
        _mem_read_kernel,
        grid=(NUM_SLOT_TILES,),
        in_specs=[
            pl.BlockSpec((BATCH, INPUT_DIM), lambda s: (0, SEQ - 1)),
            pl.BlockSpec((KEY_DIM, D_KEY), lambda s: (0, 0)),
            pl.BlockSpec((1, D_KEY), lambda s: (0, 0)),
            pl.BlockSpec((SLOT_TILE, D_KEY), lambda s: (s, 0)),
            pl.BlockSpec((SLOT_TILE, D_VAL), lambda s: (s, 0)),
            pl.BlockSpec((BATCH, HIDDEN), lambda s: (0, 0)),
            pl.BlockSpec((HIDDEN, out_W.shape[0]), lambda s: (0, 0)),
            pl.BlockSpec((D_VAL, out_W.shape[0]), lambda s: (0, 0)),
            pl.BlockSpec((1, out_W.shape[0]), lambda s: (0, 0)),
        ],
        out_specs=pl.BlockSpec((BATCH, out_W.shape[0]), lambda s: (0, 0)),
        out_shape=jax.ShapeDtypeStruct((BATCH, out_W.shape[0]), f32),
        scratch_shapes=[
            pltpu.VMEM((BATCH, D_KEY), jnp.float8_e4m3fn),
            pltpu.VMEM((SLOT_TILE, 128), bf16),
            pltpu.VMEM((BATCH, 128), f32),
        ],
    )(inputs.reshape(BATCH, SEQ * INPUT_DIM), kpw_t, kpb, mem_keys,
      mem_values, final_h, owh_t, owc_t, ob)

    return logits


# speedup vs baseline: 2.0563x; 2.0331x over previous
import functools
import math

import jax
import jax.numpy as jnp
from jax.experimental import pallas as pl
from jax.experimental.pallas import tpu as pltpu

INPUT_DIM = 128
HIDDEN = 128
KEY_DIM = 64
MAX_SLOTS = 65536
D_KEY = 64
D_VAL = 64
TEMP = 0.1
BATCH = 512
SEQ = 50
EPS = 1e-8

SLOT_TILE = 8192
NUM_SLOT_TILES = MAX_SLOTS // SLOT_TILE


def _lstm_step_kernel(x_ref, w_ref, b_ref, out_ref, xh_ref, c_ref):
    t = pl.program_id(0)

    @pl.when(t == 0)
    def _init():
        xh_ref[...] = jnp.zeros_like(xh_ref)
        c_ref[...] = jnp.zeros_like(c_ref)

    xh_ref[:, :INPUT_DIM] = x_ref[0].astype(jnp.bfloat16)
    gates = (
        jnp.dot(xh_ref[...], w_ref[...], preferred_element_type=jnp.float32)
        + b_ref[...]
    )

    def _sigmoid(x):
        return 0.5 * jnp.tanh(0.5 * x) + 0.5

    i = _sigmoid(gates[:, 0 * HIDDEN:1 * HIDDEN])
    f = _sigmoid(gates[:, 1 * HIDDEN:2 * HIDDEN])
    g = jnp.tanh(gates[:, 2 * HIDDEN:3 * HIDDEN])
    o = _sigmoid(gates[:, 3 * HIDDEN:4 * HIDDEN])
    c_new = f * c_ref[...] + i * g
    h_new = o * jnp.tanh(c_new)
    c_ref[...] = c_new
    xh_ref[:, INPUT_DIM:] = h_new.astype(jnp.bfloat16)

    @pl.when(t == SEQ - 1)
    def _fin():
        out_ref[...] = h_new


def _mem_read_kernel(cq_ref, kpw_ref, kpb_ref, keys_ref, vals_ref, fh_ref,
                     owh_ref, owc_ref, ob_ref, out_ref,
                     qa_ref, va_ref, acc_ref):
    s = pl.program_id(0)

    @pl.when(s == 0)
    def _init():
        q = (
            jnp.dot(cq_ref[...], kpw_ref[...], preferred_element_type=jnp.float32)
            + kpb_ref[...]
        )
        nrm = jnp.sqrt(jnp.sum(q * q, axis=-1, keepdims=True))
        qn = q / (nrm + EPS)
        qa_ref[...] = (qn * (math.log2(math.e) / TEMP)).astype(qa_ref.dtype)
        va_ref[D_VAL:, :] = jnp.ones((128 - D_VAL, SLOT_TILE), jnp.bfloat16)
        acc_ref[...] = jnp.zeros_like(acc_ref)

    kt = keys_ref[...]
    knrm = jnp.sqrt(jnp.sum(kt * kt, axis=0, keepdims=True))
    kn = (kt * (1.0 / (knrm + EPS))).astype(qa_ref.dtype)
    logits2 = jax.lax.dot_general(
        qa_ref[...], kn, (((1,), (0,)), ((), ())),
        preferred_element_type=jnp.float32,
    )
    p = jnp.exp2(logits2.astype(jnp.bfloat16))
    va_ref[:D_VAL, :] = vals_ref[...].astype(jnp.bfloat16)
    acc_ref[...] += jax.lax.dot_general(
        p, va_ref[...], (((1,), (1,)), ((), ())),
        preferred_element_type=jnp.float32,
    )

    @pl.when(s == NUM_SLOT_TILES - 1)
    def _fin():
        ctx = acc_ref[:, :D_VAL] / acc_ref[:, D_VAL:D_VAL + 1]
        out_ref[...] = (
            jnp.dot(fh_ref[...], owh_ref[...], preferred_element_type=jnp.float32)
            + jnp.dot(ctx, owc_ref[...], preferred_element_type=jnp.float32)
            + ob_ref[...]
        )


@functools.partial(jax.jit, static_argnames=())
def kernel(inputs, W_ih, W_hh, b_ih, b_hh, key_proj_W, key_proj_b,
           mem_keys, mem_values, out_W, out_b):
    f32 = jnp.float32
    bf16 = jnp.bfloat16
    xs = jnp.swapaxes(inputs, 0, 1)
    w_cat = jnp.concatenate([W_ih.T, W_hh.T], axis=0).astype(bf16)
    bias = (b_ih + b_hh).reshape(1, 4 * HIDDEN)

    final_h = pl.pallas_call(
        _lstm_step_kernel,
        grid=(SEQ,),
        in_specs=[
            pl.BlockSpec((1, BATCH, INPUT_DIM), lambda t: (t, 0, 0)),
            pl.BlockSpec((INPUT_DIM + HIDDEN, 4 * HIDDEN), lambda t: (0, 0)),
            pl.BlockSpec((1, 4 * HIDDEN), lambda t: (0, 0)),
        ],
        out_specs=pl.BlockSpec((BATCH, HIDDEN), lambda t: (0, 0)),
        out_shape=jax.ShapeDtypeStruct((BATCH, HIDDEN), f32),
        scratch_shapes=[
            pltpu.VMEM((BATCH, INPUT_DIM + HIDDEN), bf16),
            pltpu.VMEM((BATCH, HIDDEN), f32),
        ],
    )(xs, w_cat, bias)

    cq = inputs[:, -1, :KEY_DIM]
    kpw_t = key_proj_W.T
    kpb = key_proj_b.reshape(1, D_KEY)
    keys_t = mem_keys.T
    vals_t = mem_values.T
    owh_t = out_W[:, :HIDDEN].T
    owc_t = out_W[:, HIDDEN:].T
    ob = out_b.reshape(1, -1)

    logits = pl.pallas_call(
        _mem_read_kernel,
        grid=(NUM_SLOT_TILES,),
        in_specs=[
            pl.BlockSpec((BATCH, KEY_DIM), lambda s: (0, 0)),
            pl.BlockSpec((KEY_DIM, D_KEY), lambda s: (0, 0)),
            pl.BlockSpec((1, D_KEY), lambda s: (0, 0)),
            pl.BlockSpec((D_KEY, SLOT_TILE), lambda s: (0, s)),
            pl.BlockSpec((D_VAL, SLOT_TILE), lambda s: (0, s)),
            pl.BlockSpec((BATCH, HIDDEN), lambda s: (0, 0)),
            pl.BlockSpec((HIDDEN, out_W.shape[0]), lambda s: (0, 0)),
            pl.BlockSpec((D_VAL, out_W.shape[0]), lambda s: (0, 0)),
            pl.BlockSpec((1, out_W.shape[0]), lambda s: (0, 0)),
        ],
        out_specs=pl.BlockSpec((BATCH, out_W.shape[0]), lambda s: (0, 0)),
        out_shape=jax.ShapeDtypeStruct((BATCH, out_W.shape[0]), f32),
        scratch_shapes=[
            pltpu.VMEM((BATCH, D_KEY), jnp.float8_e4m3fn),
            pltpu.VMEM((128, SLOT_TILE), bf16),
            pltpu.VMEM((BATCH, 128), f32),
        ],
    )(cq, kpw_t, kpb, keys_t, vals_t, final_h, owh_t, owc_t, ob)

    return logits
